# baseline (speedup 1.0000x reference)
import functools

import jax
import jax.numpy as jnp
from jax import lax
from jax.experimental import pallas as pl
from jax.experimental.pallas import tpu as pltpu
from jax.experimental.pallas import tpu_sc as plsc

B, Q, C = 8, 1000, 4096
QPW = 256
QSTEP = 248
NG = QPW // 16


NSLOT = 8


def _sc_body(cl_h, out_h, buf2, rowbufs, flags, unres, sem, sem2):
    wid = lax.axis_index("s") * 2 + lax.axis_index("c")
    img = wid // 4
    qs = (wid % 4) * QSTEP

    half = QPW // 16
    cps = [
        pltpu.async_copy(
            cl_h.at[img, pl.ds(qs // 8 + i * half, half), 0],
            buf2.at[pl.ds(i * half, half), :, pl.ds(0, 128)], s)
        for i, s in ((0, sem), (1, sem2))
    ]

    lanes = lax.iota(jnp.int32, 16)

    zero16 = lanes * 0

    def char_group(gi, _):
        rows = gi * 16 + lanes
        tq = rows // 8
        qi = rows % 8
        v0 = plsc.load_gather(buf2, [tq, qi, zero16])
        accs = [v0 != v0] * 4
        for c0 in range(0, 128, 4):
            for k in range(4):
                cvec = jnp.full((16,), c0 + k, jnp.int32)
                accs[k] = jnp.logical_or(
                    accs[k], plsc.load_gather(buf2, [tq, qi, cvec]) > v0)
        acc = jnp.logical_or(jnp.logical_or(accs[0], accs[1]),
                             jnp.logical_or(accs[2], accs[3]))
        flags[pl.ds(gi * 16, 16)] = jnp.where(acc, 1.0, -1.0)
        return 0

    def straggler_dma(r, slot, s):
        return pltpu.async_copy(
            cl_h.at[img, qs // 8 + r // 8, pl.ds(1, 31), r % 8],
            rowbufs.at[slot], s)

    def issue_group(gi, cnt):
        fvec = flags[pl.ds(gi * 16, 16)]

        def issue_row(rr, cnt):
            r = gi * 16 + rr
            f = plsc.load_gather(flags, [r + lanes * 0])
            go = jnp.logical_and(f[0] < 0.0, cnt < NSLOT)

            @pl.when(go)
            def _():
                straggler_dma(r, cnt, sem)
                plsc.store_scatter(unres, [cnt + lanes * 0], r + lanes * 0,
                                   mask=lanes == 0)
            return cnt + go.astype(jnp.int32)

        return lax.cond(jnp.min(fvec) < 0.0,
                        lambda c: lax.fori_loop(0, 16, issue_row, c),
                        lambda c: c, cnt)

    cnt = 0
    for i in range(2):
        cps[i].wait()
        lax.fori_loop(i * NG // 2, (i + 1) * NG // 2, char_group, 0)
        cnt = lax.fori_loop(i * NG // 2, (i + 1) * NG // 2, issue_group, cnt)

    def drain(i, _):
        pltpu.make_async_copy(
            cl_h.at[img, 0, pl.ds(1, 31), 0], rowbufs.at[0], sem).wait()
        return 0

    lax.fori_loop(0, cnt, drain, 0)

    def scan_rowbuf(ref, i, r):
        accs = tuple(ref[i, 0, pl.ds(k * 16, 16)] for k in range(8))

        def chunk(t, a):
            return tuple(jnp.maximum(a[k], ref[i, t, pl.ds(k * 16, 16)])
                         for k in range(8))

        accs = lax.fori_loop(1, 31, chunk, accs)
        m = accs[0]
        for k in range(1, 8):
            m = jnp.maximum(m, accs[k])
        v0v = plsc.load_gather(
            buf2, [lanes * 0 + r // 8, lanes * 0 + r % 8, lanes * 0])
        val = jnp.where(jnp.max(m) > v0v[0], 1.0, 0.0) + lanes * 0.0
        plsc.store_scatter(flags, [r + lanes * 0], val, mask=lanes == 0)

    def scan_slot(i, _):
        r = plsc.load_gather(unres, [i + lanes * 0])[0]
        scan_rowbuf(rowbufs, i, r)
        return 0

    lax.fori_loop(0, cnt, scan_slot, 0)

    @pl.when(cnt >= NSLOT)
    def _():
        def resolve(r, _):
            f = plsc.load_gather(flags, [r + lanes * 0])

            @pl.when(f[0] < 0.0)
            def _():
                straggler_dma(r, 0, sem).wait()
                scan_rowbuf(rowbufs, 0, r)
            return 0

        lax.fori_loop(0, QPW, resolve, 0)

    pltpu.sync_copy(flags, out_h.at[pl.ds(img * 1024 + qs, QPW)])


def _tc_decode(ts, bb, lb, ch, blg, llg, out):
    for b in range(B):
        h = ts[0, b]
        w = ts[1, b]

        def keep(ref, b=b):
            v0 = ref[b, 0, :]
            acc = ref[b, 1, :] > v0
            for c in range(2, 16):
                acc = jnp.logical_or(acc, ref[b, c, :] > v0)
            return jnp.where(acc, 1.0, 0.0)

        fb = keep(blg)
        fl = keep(llg)
        for base, src, f in ((0, bb, fb), (4, lb, fl)):
            cx = src[b, 0, :]
            cy = src[b, 1, :]
            hw = src[b, 2, :] * 0.5
            hh = src[b, 3, :] * 0.5
            out[b, base + 0, :] = (cx - hw) * w * f
            out[b, base + 1, :] = (cy - hh) * h * f
            out[b, base + 2, :] = (cx + hw) * w * f
            out[b, base + 3, :] = (cy + hh) * h * f
        for c in range(16):
            s = h if c % 2 == 0 else w
            out[b, 8 + c, :] = ch[b, c, :] * s


def _tc_mask(o1, cf, out):
    for b in range(B):
        fc = cf[8 * b:8 * b + 8, :].reshape(1024)[:Q]
        for c in range(8):
            out[b, c, :] = o1[b, c, :] * fc


@jax.jit
def kernel(pred_block, pred_line, pred_char, pred_block_logits,
           pred_line_logits, pred_char_logits, target_sizes):
    mesh = plsc.VectorSubcoreMesh(core_axis_name="c", subcore_axis_name="s")
    sc_run = functools.partial(
        pl.kernel,
        mesh=mesh,
        compiler_params=pltpu.CompilerParams(
            needs_layout_passes=False, use_tc_tiling_on_sc=False),
        cost_estimate=pl.CostEstimate(
            flops=B * Q * C, bytes_accessed=B * Q * C * 4, transcendentals=0),
        out_type=jax.ShapeDtypeStruct((B * 1024,), jnp.float32),
        scratch_types=[
            pltpu.VMEM((QPW // 8, 8, 129), jnp.float32),
            pltpu.VMEM((NSLOT, 31, 128), jnp.float32),
            pltpu.VMEM((QPW,), jnp.float32),
            pltpu.VMEM((16,), jnp.int32),
            pltpu.SemaphoreType.DMA,
            pltpu.SemaphoreType.DMA,
        ],
    )(_sc_body)
    cflags = sc_run(
        pred_char_logits.reshape(B, Q // 8, 8, C // 128, 128)
                        .transpose(0, 1, 3, 2, 4)).reshape(B * 8, 128)

    o1 = pl.pallas_call(
        _tc_decode,
        out_shape=jax.ShapeDtypeStruct((B, 24, Q), jnp.float32),
    )(target_sizes.transpose(1, 0),
      pred_block.transpose(0, 2, 1), pred_line.transpose(0, 2, 1),
      pred_char.transpose(0, 2, 1),
      pred_block_logits.transpose(0, 2, 1),
      pred_line_logits.transpose(0, 2, 1))

    out = pl.pallas_call(
        _tc_mask,
        grid=(2,),
        in_specs=[
            pl.BlockSpec((B, 8, Q), lambda j: (0, j + 1, 0)),
            pl.BlockSpec((B * 8, 128), lambda j: (0, 0)),
        ],
        out_specs=pl.BlockSpec((B, 8, Q), lambda j: (0, j + 1, 0)),
        out_shape=jax.ShapeDtypeStruct((B, 24, Q), jnp.float32),
        input_output_aliases={0: 0},
    )(o1, cflags)
    return out.transpose(0, 2, 1)


# speedup vs baseline: 1.0010x; 1.0010x over previous
import functools

import jax
import jax.numpy as jnp
from jax import lax
from jax.experimental import pallas as pl
from jax.experimental.pallas import tpu as pltpu
from jax.experimental.pallas import tpu_sc as plsc

B, Q, C = 8, 1000, 4096
QPW = 256
QSTEP = 248
NG = QPW // 16


NSLOT = 8


def _sc_body(cl_h, out_h, buf2, rowbufs, flags, unres, sem, sem2):
    wid = lax.axis_index("s") * 2 + lax.axis_index("c")
    img = wid // 4
    qs = (wid % 4) * QSTEP

    half = QPW // 16
    cps = [
        pltpu.async_copy(
            cl_h.at[img, pl.ds(qs // 8 + i * half, half), 0],
            buf2.at[pl.ds(i * half, half), :, pl.ds(0, 128)], s)
        for i, s in ((0, sem), (1, sem2))
    ]

    lanes = lax.iota(jnp.int32, 16)

    zero16 = lanes * 0

    def char_group(gi, _):
        rows = gi * 16 + lanes
        tq = rows // 8
        qi = rows % 8
        v0 = plsc.load_gather(buf2, [tq, qi, zero16])
        accs = [v0 != v0] * 4
        for c0 in range(0, 128, 4):
            for k in range(4):
                cvec = jnp.full((16,), c0 + k, jnp.int32)
                accs[k] = jnp.logical_or(
                    accs[k], plsc.load_gather(buf2, [tq, qi, cvec]) > v0)
        acc = jnp.logical_or(jnp.logical_or(accs[0], accs[1]),
                             jnp.logical_or(accs[2], accs[3]))
        flags[pl.ds(gi * 16, 16)] = jnp.where(acc, 1.0, -1.0)
        return 0

    def straggler_dma(r, slot, s):
        return pltpu.async_copy(
            cl_h.at[img, qs // 8 + r // 8, pl.ds(1, 31), r % 8],
            rowbufs.at[slot], s)

    def issue_group(gi, cnt):
        fvec = flags[pl.ds(gi * 16, 16)]

        def issue_row(rr, cnt):
            r = gi * 16 + rr
            f = plsc.load_gather(flags, [r + lanes * 0])
            go = jnp.logical_and(f[0] < 0.0, cnt < NSLOT)

            @pl.when(go)
            def _():
                straggler_dma(r, cnt, sem)
                plsc.store_scatter(unres, [cnt + lanes * 0], r + lanes * 0,
                                   mask=lanes == 0)
            return cnt + go.astype(jnp.int32)

        return lax.cond(jnp.min(fvec) < 0.0,
                        lambda c: lax.fori_loop(0, 16, issue_row, c),
                        lambda c: c, cnt)

    cnt = 0
    for i in range(2):
        cps[i].wait()
        lax.fori_loop(i * NG // 2, (i + 1) * NG // 2, char_group, 0)
        cnt = lax.fori_loop(i * NG // 2, (i + 1) * NG // 2, issue_group, cnt)

    def drain(i, _):
        pltpu.make_async_copy(
            cl_h.at[img, 0, pl.ds(1, 31), 0], rowbufs.at[0], sem).wait()
        return 0

    lax.fori_loop(0, cnt, drain, 0)

    def scan_rowbuf(ref, i, r):
        accs = tuple(ref[i, 0, pl.ds(k * 16, 16)] for k in range(8))

        def chunk(t, a):
            return tuple(jnp.maximum(a[k], ref[i, t, pl.ds(k * 16, 16)])
                         for k in range(8))

        accs = lax.fori_loop(1, 31, chunk, accs)
        m = accs[0]
        for k in range(1, 8):
            m = jnp.maximum(m, accs[k])
        v0v = plsc.load_gather(
            buf2, [lanes * 0 + r // 8, lanes * 0 + r % 8, lanes * 0])
        val = jnp.where(jnp.max(m) > v0v[0], 1.0, 0.0) + lanes * 0.0
        plsc.store_scatter(flags, [r + lanes * 0], val, mask=lanes == 0)

    def scan_slot(i, _):
        r = plsc.load_gather(unres, [i + lanes * 0])[0]
        scan_rowbuf(rowbufs, i, r)
        return 0

    lax.fori_loop(0, cnt, scan_slot, 0)

    @pl.when(cnt >= NSLOT)
    def _():
        def resolve(r, _):
            f = plsc.load_gather(flags, [r + lanes * 0])

            @pl.when(f[0] < 0.0)
            def _():
                straggler_dma(r, 0, sem).wait()
                scan_rowbuf(rowbufs, 0, r)
            return 0

        lax.fori_loop(0, QPW, resolve, 0)

    pltpu.sync_copy(flags, out_h.at[pl.ds(img * 1024 + qs, QPW)])


def _tc_decode(ts, bb, lb, ch, blg, llg, out):
    for b in range(B):
        h = ts[0, b]
        w = ts[1, b]

        def keep(ref, b=b):
            v0 = ref[b, 0, :]
            acc = ref[b, 1, :] > v0
            for c in range(2, 16):
                acc = jnp.logical_or(acc, ref[b, c, :] > v0)
            return jnp.where(acc, 1.0, 0.0)

        fb = keep(blg)
        fl = keep(llg)
        for base, src, f in ((0, bb, fb), (4, lb, fl)):
            cx = src[b, 0, :]
            cy = src[b, 1, :]
            hw = src[b, 2, :] * 0.5
            hh = src[b, 3, :] * 0.5
            out[b, base + 0, :] = (cx - hw) * w * f
            out[b, base + 1, :] = (cy - hh) * h * f
            out[b, base + 2, :] = (cx + hw) * w * f
            out[b, base + 3, :] = (cy + hh) * h * f
        for c in range(16):
            s = h if c % 2 == 0 else w
            out[b, 8 + c, :] = ch[b, c, :] * s


def _tc_mask(o1, cf, out):
    for b in range(B):
        fc = cf[8 * b:8 * b + 8, :].reshape(1024)[:Q]
        for c in range(8):
            out[b, c, :] = o1[b, c, :] * fc


@jax.jit
def kernel(pred_block, pred_line, pred_char, pred_block_logits,
           pred_line_logits, pred_char_logits, target_sizes):
    mesh = plsc.VectorSubcoreMesh(core_axis_name="c", subcore_axis_name="s")
    sc_run = functools.partial(
        pl.kernel,
        mesh=mesh,
        compiler_params=pltpu.CompilerParams(
            needs_layout_passes=False, use_tc_tiling_on_sc=False),
        out_type=jax.ShapeDtypeStruct((B * 1024,), jnp.float32),
        scratch_types=[
            pltpu.VMEM((QPW // 8, 8, 129), jnp.float32),
            pltpu.VMEM((NSLOT, 31, 128), jnp.float32),
            pltpu.VMEM((QPW,), jnp.float32),
            pltpu.VMEM((16,), jnp.int32),
            pltpu.SemaphoreType.DMA,
            pltpu.SemaphoreType.DMA,
        ],
    )(_sc_body)
    cflags = sc_run(
        pred_char_logits.reshape(B, Q // 8, 8, C // 128, 128)
                        .transpose(0, 1, 3, 2, 4)).reshape(B * 8, 128)

    o1 = pl.pallas_call(
        _tc_decode,
        out_shape=jax.ShapeDtypeStruct((B, 24, Q), jnp.float32),
    )(target_sizes.transpose(1, 0),
      pred_block.transpose(0, 2, 1), pred_line.transpose(0, 2, 1),
      pred_char.transpose(0, 2, 1),
      pred_block_logits.transpose(0, 2, 1),
      pred_line_logits.transpose(0, 2, 1))

    out = pl.pallas_call(
        _tc_mask,
        grid=(2,),
        in_specs=[
            pl.BlockSpec((B, 8, Q), lambda j: (0, j + 1, 0)),
            pl.BlockSpec((B * 8, 128), lambda j: (0, 0)),
        ],
        out_specs=pl.BlockSpec((B, 8, Q), lambda j: (0, j + 1, 0)),
        out_shape=jax.ShapeDtypeStruct((B, 24, Q), jnp.float32),
        input_output_aliases={0: 0},
    )(o1, cflags)
    return out.transpose(0, 2, 1)


# speedup vs baseline: 1.0019x; 1.0009x over previous
import functools

import jax
import jax.numpy as jnp
from jax import lax
from jax.experimental import pallas as pl
from jax.experimental.pallas import tpu as pltpu
from jax.experimental.pallas import tpu_sc as plsc

B, Q, C = 8, 1000, 4096
QPW = 256
QSTEP = 248
NG = QPW // 16


NSLOT = 8


def _sc_body(cl_h, out_h, buf2, rowbufs, flags, unres, sem, sem2):
    wid = lax.axis_index("s") * 2 + lax.axis_index("c")
    img = wid // 4
    qs = (wid % 4) * QSTEP

    half = QPW // 16
    cps = [
        pltpu.async_copy(
            cl_h.at[img, pl.ds(qs // 8 + i * half, half), 0],
            buf2.at[pl.ds(i * half, half), :, pl.ds(0, 128)], s)
        for i, s in ((0, sem), (1, sem2))
    ]

    lanes = lax.iota(jnp.int32, 16)

    zero16 = lanes * 0

    def char_group(gi, _):
        rows = gi * 16 + lanes
        tq = rows // 8
        qi = rows % 8
        v0 = plsc.load_gather(buf2, [tq, qi, zero16])
        accs = [v0 != v0] * 8
        for c0 in range(0, 128, 8):
            for k in range(8):
                cvec = jnp.full((16,), c0 + k, jnp.int32)
                accs[k] = jnp.logical_or(
                    accs[k], plsc.load_gather(buf2, [tq, qi, cvec]) > v0)
        acc = accs[0]
        for k in range(1, 8):
            acc = jnp.logical_or(acc, accs[k])
        flags[pl.ds(gi * 16, 16)] = jnp.where(acc, 1.0, -1.0)
        return 0

    def straggler_dma(r, slot, s):
        return pltpu.async_copy(
            cl_h.at[img, qs // 8 + r // 8, pl.ds(1, 31), r % 8],
            rowbufs.at[slot], s)

    def issue_group(gi, cnt):
        fvec = flags[pl.ds(gi * 16, 16)]

        def issue_row(rr, cnt):
            r = gi * 16 + rr
            f = plsc.load_gather(flags, [r + lanes * 0])
            go = jnp.logical_and(f[0] < 0.0, cnt < NSLOT)

            @pl.when(go)
            def _():
                straggler_dma(r, cnt, sem)
                plsc.store_scatter(unres, [cnt + lanes * 0], r + lanes * 0,
                                   mask=lanes == 0)
            return cnt + go.astype(jnp.int32)

        return lax.cond(jnp.min(fvec) < 0.0,
                        lambda c: lax.fori_loop(0, 16, issue_row, c),
                        lambda c: c, cnt)

    cnt = 0
    for i in range(2):
        cps[i].wait()
        lax.fori_loop(i * NG // 2, (i + 1) * NG // 2, char_group, 0)
        cnt = lax.fori_loop(i * NG // 2, (i + 1) * NG // 2, issue_group, cnt)

    def drain(i, _):
        pltpu.make_async_copy(
            cl_h.at[img, 0, pl.ds(1, 31), 0], rowbufs.at[0], sem).wait()
        return 0

    lax.fori_loop(0, cnt, drain, 0)

    def scan_rowbuf(ref, i, r):
        accs = tuple(ref[i, 0, pl.ds(k * 16, 16)] for k in range(8))

        def chunk(t, a):
            return tuple(jnp.maximum(a[k], ref[i, t, pl.ds(k * 16, 16)])
                         for k in range(8))

        accs = lax.fori_loop(1, 31, chunk, accs)
        m = accs[0]
        for k in range(1, 8):
            m = jnp.maximum(m, accs[k])
        v0v = plsc.load_gather(
            buf2, [lanes * 0 + r // 8, lanes * 0 + r % 8, lanes * 0])
        val = jnp.where(jnp.max(m) > v0v[0], 1.0, 0.0) + lanes * 0.0
        plsc.store_scatter(flags, [r + lanes * 0], val, mask=lanes == 0)

    def scan_slot(i, _):
        r = plsc.load_gather(unres, [i + lanes * 0])[0]
        scan_rowbuf(rowbufs, i, r)
        return 0

    lax.fori_loop(0, cnt, scan_slot, 0)

    @pl.when(cnt >= NSLOT)
    def _():
        def resolve(r, _):
            f = plsc.load_gather(flags, [r + lanes * 0])

            @pl.when(f[0] < 0.0)
            def _():
                straggler_dma(r, 0, sem).wait()
                scan_rowbuf(rowbufs, 0, r)
            return 0

        lax.fori_loop(0, QPW, resolve, 0)

    pltpu.sync_copy(flags, out_h.at[pl.ds(img * 1024 + qs, QPW)])


def _tc_decode(ts, bb, lb, ch, blg, llg, out):
    for b in range(B):
        h = ts[0, b]
        w = ts[1, b]

        def keep(ref, b=b):
            v0 = ref[b, 0, :]
            acc = ref[b, 1, :] > v0
            for c in range(2, 16):
                acc = jnp.logical_or(acc, ref[b, c, :] > v0)
            return jnp.where(acc, 1.0, 0.0)

        fb = keep(blg)
        fl = keep(llg)
        for base, src, f in ((0, bb, fb), (4, lb, fl)):
            cx = src[b, 0, :]
            cy = src[b, 1, :]
            hw = src[b, 2, :] * 0.5
            hh = src[b, 3, :] * 0.5
            out[b, base + 0, :] = (cx - hw) * w * f
            out[b, base + 1, :] = (cy - hh) * h * f
            out[b, base + 2, :] = (cx + hw) * w * f
            out[b, base + 3, :] = (cy + hh) * h * f
        for c in range(16):
            s = h if c % 2 == 0 else w
            out[b, 8 + c, :] = ch[b, c, :] * s


def _tc_mask(o1, cf, out):
    for b in range(B):
        fc = cf[8 * b:8 * b + 8, :].reshape(1024)[:Q]
        for c in range(8):
            out[b, c, :] = o1[b, c, :] * fc


@jax.jit
def kernel(pred_block, pred_line, pred_char, pred_block_logits,
           pred_line_logits, pred_char_logits, target_sizes):
    mesh = plsc.VectorSubcoreMesh(core_axis_name="c", subcore_axis_name="s")
    sc_run = functools.partial(
        pl.kernel,
        mesh=mesh,
        compiler_params=pltpu.CompilerParams(
            needs_layout_passes=False, use_tc_tiling_on_sc=False),
        out_type=jax.ShapeDtypeStruct((B * 1024,), jnp.float32),
        scratch_types=[
            pltpu.VMEM((QPW // 8, 8, 129), jnp.float32),
            pltpu.VMEM((NSLOT, 31, 128), jnp.float32),
            pltpu.VMEM((QPW,), jnp.float32),
            pltpu.VMEM((16,), jnp.int32),
            pltpu.SemaphoreType.DMA,
            pltpu.SemaphoreType.DMA,
        ],
    )(_sc_body)
    cflags = sc_run(
        pred_char_logits.reshape(B, Q // 8, 8, C // 128, 128)
                        .transpose(0, 1, 3, 2, 4)).reshape(B * 8, 128)

    o1 = pl.pallas_call(
        _tc_decode,
        out_shape=jax.ShapeDtypeStruct((B, 24, Q), jnp.float32),
    )(target_sizes.transpose(1, 0),
      pred_block.transpose(0, 2, 1), pred_line.transpose(0, 2, 1),
      pred_char.transpose(0, 2, 1),
      pred_block_logits.transpose(0, 2, 1),
      pred_line_logits.transpose(0, 2, 1))

    out = pl.pallas_call(
        _tc_mask,
        grid=(2,),
        in_specs=[
            pl.BlockSpec((B, 8, Q), lambda j: (0, j + 1, 0)),
            pl.BlockSpec((B * 8, 128), lambda j: (0, 0)),
        ],
        out_specs=pl.BlockSpec((B, 8, Q), lambda j: (0, j + 1, 0)),
        out_shape=jax.ShapeDtypeStruct((B, 24, Q), jnp.float32),
        input_output_aliases={0: 0},
    )(o1, cflags)
    return out.transpose(0, 2, 1)
